# Initial kernel scaffold; baseline (speedup 1.0000x reference)
import jax
import jax.numpy as jnp
from jax.experimental import pallas as pl


def kernel(x, edge_index, edge_weight, batch, W1, b1, W2, b2, W3, b3, gamma, beta):
    raise NotImplementedError("write your pallas kernel here")


# speedup vs baseline: 5.7253x; 5.7253x over previous
import functools

import jax
import jax.numpy as jnp
from jax import lax
from jax.experimental import pallas as pl
from jax.experimental.pallas import tpu as pltpu
from jax.experimental.pallas import tpu_sc as plsc

N = 10000
E = 320000
D_IN = 128
D_H = 256
DHH = 128
G_GRAPHS = 64

HALF = N // 2
ROWS = 320
ACC_ROWS = 16 * ROWS
CHUNK = 80
TILE_EDGES = E // 16
TILE_CHUNKS = TILE_EDGES // CHUNK

BLK = 1000
GRID = N // BLK

_mesh = plsc.VectorSubcoreMesh(core_axis_name="c", subcore_axis_name="s")


def _remap_dst(dst_v, half_lo):
    for j in range(CHUNK // 16):
        d = dst_v[pl.ds(j * 16, 16)]
        dl = d - half_lo
        ok = (dl >= 0) & (dl < HALF)
        trash = HALF + (d & 63)
        dst_v[pl.ds(j * 16, 16)] = jnp.where(ok, dl, trash)


def _zero_acc(zeros_hbm, acc, s):
    pltpu.sync_copy(zeros_hbm, acc.at[pl.ds(s * ROWS, ROWS)])


def _write_out(acc, out_hbm, c, s):
    @pl.when(s < 15)
    def _():
        pltpu.sync_copy(acc.at[pl.ds(s * ROWS, ROWS)],
                        out_hbm.at[pl.ds(c * HALF + s * ROWS, ROWS)])

    @pl.when(s == 15)
    def _():
        rem = HALF - 15 * ROWS
        pltpu.sync_copy(acc.at[pl.ds(15 * ROWS, rem)],
                        out_hbm.at[pl.ds(c * HALF + 15 * ROWS, rem)])


@functools.partial(
    pl.kernel,
    mesh=_mesh,
    out_type=jax.ShapeDtypeStruct((N, DHH), jnp.float32),
    scratch_types=[
        pltpu.VMEM((CHUNK,), jnp.int32),
        pltpu.VMEM((CHUNK, DHH), jnp.float32),
        pltpu.VMEM_SHARED((ACC_ROWS, DHH), jnp.float32),
    ],
)
def _deg_sc(dst_hbm, ones_hbm, zeros_hbm, deg_hbm, dst_v, ones_v, dacc):
    c = lax.axis_index("c")
    s = lax.axis_index("s")
    _zero_acc(zeros_hbm, dacc, s)
    pltpu.sync_copy(ones_hbm, ones_v)
    plsc.subcore_barrier()
    base0 = s * TILE_EDGES
    half_lo = c * HALF

    def body(i, carry):
        pltpu.sync_copy(dst_hbm.at[pl.ds(base0 + i * CHUNK, CHUNK)], dst_v)
        _remap_dst(dst_v, half_lo)
        pltpu.sync_copy(ones_v, dacc.at[dst_v], add=True)
        return carry

    lax.fori_loop(0, TILE_CHUNKS, body, 0)
    plsc.subcore_barrier()
    _write_out(dacc, deg_hbm, c, s)


@functools.partial(
    pl.kernel,
    mesh=_mesh,
    out_type=(jax.ShapeDtypeStruct((N, DHH), jnp.float32),
              jax.ShapeDtypeStruct((N, DHH), jnp.float32)),
    scratch_types=[
        pltpu.VMEM((CHUNK,), jnp.int32),
        pltpu.VMEM((CHUNK,), jnp.int32),
        pltpu.VMEM((CHUNK, DHH), jnp.float32),
        pltpu.VMEM((CHUNK, DHH), jnp.float32),
        pltpu.VMEM_SHARED((ACC_ROWS, DHH), jnp.float32),
        pltpu.VMEM_SHARED((ACC_ROWS, DHH), jnp.float32),
        pltpu.SemaphoreType.DMA,
        pltpu.SemaphoreType.DMA,
    ],
)
def _prop_sc(glo_hbm, ghi_hbm, src_hbm, dst_hbm, zeros_hbm, plo_hbm, phi_hbm,
             src_v, dst_v, buf_lo, buf_hi, acc_lo, acc_hi, sem_lo, sem_hi):
    c = lax.axis_index("c")
    s = lax.axis_index("s")
    _zero_acc(zeros_hbm, acc_lo, s)
    _zero_acc(zeros_hbm, acc_hi, s)
    plsc.subcore_barrier()
    base0 = s * TILE_EDGES
    half_lo = c * HALF

    def body(i, carry):
        base = base0 + i * CHUNK
        pltpu.sync_copy(dst_hbm.at[pl.ds(base, CHUNK)], dst_v)
        _remap_dst(dst_v, half_lo)
        pltpu.sync_copy(src_hbm.at[pl.ds(base, CHUNK)], src_v)
        lo = pltpu.async_copy(glo_hbm.at[src_v], buf_lo, sem_lo)
        hi = pltpu.async_copy(ghi_hbm.at[src_v], buf_hi, sem_hi)
        lo.wait()
        pltpu.sync_copy(buf_lo, acc_lo.at[dst_v], add=True)
        hi.wait()
        pltpu.sync_copy(buf_hi, acc_hi.at[dst_v], add=True)
        return carry

    lax.fori_loop(0, TILE_CHUNKS, body, 0)
    plsc.subcore_barrier()
    _write_out(acc_lo, plo_hbm, c, s)
    _write_out(acc_hi, phi_hbm, c, s)


def _mm1_body(deg_ref, x_ref, w_ref, glo_ref, ghi_ref, dinv_ref):
    dinv = lax.rsqrt(deg_ref[:, :1] + 1.0)
    g = dinv * jnp.dot(x_ref[...], w_ref[...],
                       preferred_element_type=jnp.float32)
    glo_ref[...] = g[:, :DHH]
    ghi_ref[...] = g[:, DHH:]
    dinv_ref[...] = dinv


def _mid_body(plo_ref, phi_ref, glo_ref, ghi_ref, dinv_ref, b_ref, w_ref,
              olo_ref, ohi_ref):
    dinv = dinv_ref[...]
    pg = jnp.concatenate([plo_ref[...] + glo_ref[...],
                          phi_ref[...] + ghi_ref[...]], axis=1)
    h = jnp.maximum(dinv * pg + b_ref[...], 0.0)
    g = dinv * jnp.dot(h, w_ref[...], preferred_element_type=jnp.float32)
    olo_ref[...] = g[:, :DHH]
    ohi_ref[...] = g[:, DHH:]


def _fin_body(plo_ref, phi_ref, glo_ref, ghi_ref, dinv_ref, b_ref, batch_ref,
              gamma_ref, beta_ref, out_ref, pool_acc, cnt_acc):
    i = pl.program_id(0)

    @pl.when(i == 0)
    def _():
        pool_acc[...] = jnp.zeros_like(pool_acc)
        cnt_acc[...] = jnp.zeros_like(cnt_acc)

    dinv = dinv_ref[...]
    pg = jnp.concatenate([plo_ref[...] + glo_ref[...],
                          phi_ref[...] + ghi_ref[...]], axis=1)
    h = jnp.maximum(dinv * pg + b_ref[...], 0.0)
    ids = lax.broadcasted_iota(jnp.int32, (G_GRAPHS, BLK), 0)
    onehot = (ids == batch_ref[0]).astype(jnp.float32)
    pool_acc[...] += jnp.dot(onehot, h, preferred_element_type=jnp.float32)
    cnt_acc[...] += jnp.sum(onehot, axis=1, keepdims=True)

    @pl.when(i == GRID - 1)
    def _():
        pooled = pool_acc[...] / jnp.maximum(cnt_acc[...], 1.0)
        mu = jnp.mean(pooled, axis=-1, keepdims=True)
        var = jnp.mean((pooled - mu) ** 2, axis=-1, keepdims=True)
        out_ref[...] = ((pooled - mu) * lax.rsqrt(var + 1e-5)
                        * gamma_ref[...] + beta_ref[...])


_ROW_SPEC = pl.BlockSpec((BLK, DHH), lambda i: (i, 0))
_DINV_SPEC = pl.BlockSpec((BLK, 1), lambda i: (i, 0))
_VEC_SPEC = pl.BlockSpec((1, D_H), lambda i: (0, 0))


def _mm1(deg, x, w1):
    return pl.pallas_call(
        _mm1_body,
        grid=(GRID,),
        in_specs=[
            _ROW_SPEC,
            pl.BlockSpec((BLK, D_IN), lambda i: (i, 0)),
            pl.BlockSpec((D_IN, D_H), lambda i: (0, 0)),
        ],
        out_specs=[_ROW_SPEC, _ROW_SPEC, _DINV_SPEC],
        out_shape=[
            jax.ShapeDtypeStruct((N, DHH), jnp.float32),
            jax.ShapeDtypeStruct((N, DHH), jnp.float32),
            jax.ShapeDtypeStruct((N, 1), jnp.float32),
        ],
    )(deg, x, w1)


def _mid(plo, phi, glo, ghi, dinv, b, w):
    return pl.pallas_call(
        _mid_body,
        grid=(GRID,),
        in_specs=[
            _ROW_SPEC, _ROW_SPEC, _ROW_SPEC, _ROW_SPEC, _DINV_SPEC,
            _VEC_SPEC,
            pl.BlockSpec((D_H, D_H), lambda i: (0, 0)),
        ],
        out_specs=[_ROW_SPEC, _ROW_SPEC],
        out_shape=[
            jax.ShapeDtypeStruct((N, DHH), jnp.float32),
            jax.ShapeDtypeStruct((N, DHH), jnp.float32),
        ],
    )(plo, phi, glo, ghi, dinv, b, w)


def _fin(plo, phi, glo, ghi, dinv, b, batch_r, gamma, beta):
    return pl.pallas_call(
        _fin_body,
        grid=(GRID,),
        in_specs=[
            _ROW_SPEC, _ROW_SPEC, _ROW_SPEC, _ROW_SPEC, _DINV_SPEC,
            _VEC_SPEC,
            pl.BlockSpec((1, 1, BLK), lambda i: (i, 0, 0)),
            _VEC_SPEC, _VEC_SPEC,
        ],
        out_specs=pl.BlockSpec((G_GRAPHS, D_H), lambda i: (0, 0)),
        out_shape=jax.ShapeDtypeStruct((G_GRAPHS, D_H), jnp.float32),
        scratch_shapes=[
            pltpu.VMEM((G_GRAPHS, D_H), jnp.float32),
            pltpu.VMEM((G_GRAPHS, 1), jnp.float32),
        ],
    )(plo, phi, glo, ghi, dinv, b, batch_r, gamma, beta)


def kernel(x, edge_index, edge_weight, batch, W1, b1, W2, b2, W3, b3,
           gamma, beta):
    src = edge_index[0]
    dst = edge_index[1]
    ones128 = jnp.ones((CHUNK, DHH), jnp.float32)
    zeros128 = jnp.zeros((ROWS, DHH), jnp.float32)

    deg = _deg_sc(dst, ones128, zeros128)
    glo1, ghi1, dinv = _mm1(deg, x, W1)
    plo1, phi1 = _prop_sc(glo1, ghi1, src, dst, zeros128)
    glo2, ghi2 = _mid(plo1, phi1, glo1, ghi1, dinv, b1.reshape(1, D_H), W2)
    plo2, phi2 = _prop_sc(glo2, ghi2, src, dst, zeros128)
    glo3, ghi3 = _mid(plo2, phi2, glo2, ghi2, dinv, b2.reshape(1, D_H), W3)
    plo3, phi3 = _prop_sc(glo3, ghi3, src, dst, zeros128)
    return _fin(plo3, phi3, glo3, ghi3, dinv, b3.reshape(1, D_H),
                batch.reshape(GRID, 1, BLK),
                gamma.reshape(1, D_H), beta.reshape(1, D_H))


# speedup vs baseline: 8.7352x; 1.5257x over previous
import functools

import jax
import jax.numpy as jnp
from jax import lax
from jax.experimental import pallas as pl
from jax.experimental.pallas import tpu as pltpu
from jax.experimental.pallas import tpu_sc as plsc

N = 10000
E = 320000
D_IN = 128
D_H = 256
DHH = 128
G_GRAPHS = 64

HALF = N // 2
ROWS = 320
ACC_ROWS = 16 * ROWS
CHUNK = 80
TILE_EDGES = E // 16
TILE_CHUNKS = TILE_EDGES // CHUNK

BLK = 1000
GRID = N // BLK

_mesh = plsc.VectorSubcoreMesh(core_axis_name="c", subcore_axis_name="s")


def _remap_dst(dst_v, half_lo):
    for j in range(CHUNK // 16):
        d = dst_v[pl.ds(j * 16, 16)]
        dl = d - half_lo
        ok = (dl >= 0) & (dl < HALF)
        trash = HALF + (d & 63)
        dst_v[pl.ds(j * 16, 16)] = jnp.where(ok, dl, trash)


def _zero_acc(zeros_hbm, acc, s):
    pltpu.sync_copy(zeros_hbm, acc.at[pl.ds(s * ROWS, ROWS)])


def _write_out(acc, out_hbm, c, s):
    @pl.when(s < 15)
    def _():
        pltpu.sync_copy(acc.at[pl.ds(s * ROWS, ROWS)],
                        out_hbm.at[pl.ds(c * HALF + s * ROWS, ROWS)])

    @pl.when(s == 15)
    def _():
        rem = HALF - 15 * ROWS
        pltpu.sync_copy(acc.at[pl.ds(15 * ROWS, rem)],
                        out_hbm.at[pl.ds(c * HALF + 15 * ROWS, rem)])


@functools.partial(
    pl.kernel,
    mesh=_mesh,
    out_type=jax.ShapeDtypeStruct((N, DHH), jnp.float32),
    scratch_types=[
        pltpu.VMEM((CHUNK,), jnp.int32),
        pltpu.VMEM((CHUNK, DHH), jnp.float32),
        pltpu.VMEM_SHARED((ACC_ROWS, DHH), jnp.float32),
    ],
)
def _deg_sc(dst_hbm, ones_hbm, zeros_hbm, deg_hbm, dst_v, ones_v, dacc):
    c = lax.axis_index("c")
    s = lax.axis_index("s")
    _zero_acc(zeros_hbm, dacc, s)
    pltpu.sync_copy(ones_hbm, ones_v)
    plsc.subcore_barrier()
    base0 = s * TILE_EDGES
    half_lo = c * HALF

    def body(i, carry):
        pltpu.sync_copy(dst_hbm.at[pl.ds(base0 + i * CHUNK, CHUNK)], dst_v)
        _remap_dst(dst_v, half_lo)
        pltpu.sync_copy(ones_v, dacc.at[dst_v], add=True)
        return carry

    lax.fori_loop(0, TILE_CHUNKS, body, 0)
    plsc.subcore_barrier()
    _write_out(dacc, deg_hbm, c, s)


@functools.partial(
    pl.kernel,
    mesh=_mesh,
    out_type=(jax.ShapeDtypeStruct((N, DHH), jnp.float32),
              jax.ShapeDtypeStruct((N, DHH), jnp.float32)),
    scratch_types=[
        pltpu.VMEM((CHUNK,), jnp.int32),
        pltpu.VMEM((CHUNK,), jnp.int32),
        pltpu.VMEM((CHUNK,), jnp.int32),
        pltpu.VMEM((CHUNK,), jnp.int32),
        pltpu.VMEM((CHUNK, DHH), jnp.float32),
        pltpu.VMEM((CHUNK, DHH), jnp.float32),
        pltpu.VMEM((CHUNK, DHH), jnp.float32),
        pltpu.VMEM((CHUNK, DHH), jnp.float32),
        pltpu.VMEM_SHARED((ACC_ROWS, DHH), jnp.float32),
        pltpu.VMEM_SHARED((ACC_ROWS, DHH), jnp.float32),
        pltpu.SemaphoreType.DMA,
        pltpu.SemaphoreType.DMA,
        pltpu.SemaphoreType.DMA,
        pltpu.SemaphoreType.DMA,
    ],
)
def _prop_sc(glo_hbm, ghi_hbm, src_hbm, dst_hbm, zeros_hbm, plo_hbm, phi_hbm,
             src0, dst0, src1, dst1, lo0, hi0, lo1, hi1,
             acc_lo, acc_hi, sl0, sh0, sl1, sh1):
    c = lax.axis_index("c")
    s = lax.axis_index("s")
    _zero_acc(zeros_hbm, acc_lo, s)
    _zero_acc(zeros_hbm, acc_hi, s)
    plsc.subcore_barrier()
    base0 = s * TILE_EDGES
    half_lo = c * HALF

    def stage(chunk_i, src_v, dst_v, blo, bhi, semlo, semhi):
        base = base0 + chunk_i * CHUNK
        pltpu.sync_copy(dst_hbm.at[pl.ds(base, CHUNK)], dst_v)
        _remap_dst(dst_v, half_lo)
        pltpu.sync_copy(src_hbm.at[pl.ds(base, CHUNK)], src_v)
        pltpu.async_copy(glo_hbm.at[src_v], blo, semlo)
        pltpu.async_copy(ghi_hbm.at[src_v], bhi, semhi)

    def consume(src_v, dst_v, blo, bhi, semlo, semhi):
        pltpu.make_async_copy(glo_hbm.at[src_v], blo, semlo).wait()
        pltpu.sync_copy(blo, acc_lo.at[dst_v], add=True)
        pltpu.make_async_copy(ghi_hbm.at[src_v], bhi, semhi).wait()
        pltpu.sync_copy(bhi, acc_hi.at[dst_v], add=True)

    stage(0, src0, dst0, lo0, hi0, sl0, sh0)

    def body(j, carry):
        stage(2 * j + 1, src1, dst1, lo1, hi1, sl1, sh1)
        consume(src0, dst0, lo0, hi0, sl0, sh0)

        @pl.when(j < TILE_CHUNKS // 2 - 1)
        def _():
            stage(2 * j + 2, src0, dst0, lo0, hi0, sl0, sh0)

        consume(src1, dst1, lo1, hi1, sl1, sh1)
        return carry

    lax.fori_loop(0, TILE_CHUNKS // 2, body, 0)
    plsc.subcore_barrier()
    _write_out(acc_lo, plo_hbm, c, s)
    _write_out(acc_hi, phi_hbm, c, s)


def _mm1_body(deg_ref, x_ref, w_ref, glo_ref, ghi_ref, dinv_ref):
    dinv = lax.rsqrt(deg_ref[:, :1] + 1.0)
    g = dinv * jnp.dot(x_ref[...], w_ref[...],
                       preferred_element_type=jnp.float32)
    glo_ref[...] = g[:, :DHH]
    ghi_ref[...] = g[:, DHH:]
    dinv_ref[...] = dinv


def _mid_body(plo_ref, phi_ref, glo_ref, ghi_ref, dinv_ref, b_ref, w_ref,
              olo_ref, ohi_ref):
    dinv = dinv_ref[...]
    pg = jnp.concatenate([plo_ref[...] + glo_ref[...],
                          phi_ref[...] + ghi_ref[...]], axis=1)
    h = jnp.maximum(dinv * pg + b_ref[...], 0.0)
    g = dinv * jnp.dot(h, w_ref[...], preferred_element_type=jnp.float32)
    olo_ref[...] = g[:, :DHH]
    ohi_ref[...] = g[:, DHH:]


def _fin_body(plo_ref, phi_ref, glo_ref, ghi_ref, dinv_ref, b_ref, batch_ref,
              gamma_ref, beta_ref, out_ref, pool_acc, cnt_acc):
    i = pl.program_id(0)

    @pl.when(i == 0)
    def _():
        pool_acc[...] = jnp.zeros_like(pool_acc)
        cnt_acc[...] = jnp.zeros_like(cnt_acc)

    dinv = dinv_ref[...]
    pg = jnp.concatenate([plo_ref[...] + glo_ref[...],
                          phi_ref[...] + ghi_ref[...]], axis=1)
    h = jnp.maximum(dinv * pg + b_ref[...], 0.0)
    ids = lax.broadcasted_iota(jnp.int32, (G_GRAPHS, BLK), 0)
    onehot = (ids == batch_ref[0]).astype(jnp.float32)
    pool_acc[...] += jnp.dot(onehot, h, preferred_element_type=jnp.float32)
    cnt_acc[...] += jnp.sum(onehot, axis=1, keepdims=True)

    @pl.when(i == GRID - 1)
    def _():
        pooled = pool_acc[...] / jnp.maximum(cnt_acc[...], 1.0)
        mu = jnp.mean(pooled, axis=-1, keepdims=True)
        var = jnp.mean((pooled - mu) ** 2, axis=-1, keepdims=True)
        out_ref[...] = ((pooled - mu) * lax.rsqrt(var + 1e-5)
                        * gamma_ref[...] + beta_ref[...])


_ROW_SPEC = pl.BlockSpec((BLK, DHH), lambda i: (i, 0))
_DINV_SPEC = pl.BlockSpec((BLK, 1), lambda i: (i, 0))
_VEC_SPEC = pl.BlockSpec((1, D_H), lambda i: (0, 0))


def _mm1(deg, x, w1):
    return pl.pallas_call(
        _mm1_body,
        grid=(GRID,),
        in_specs=[
            _ROW_SPEC,
            pl.BlockSpec((BLK, D_IN), lambda i: (i, 0)),
            pl.BlockSpec((D_IN, D_H), lambda i: (0, 0)),
        ],
        out_specs=[_ROW_SPEC, _ROW_SPEC, _DINV_SPEC],
        out_shape=[
            jax.ShapeDtypeStruct((N, DHH), jnp.float32),
            jax.ShapeDtypeStruct((N, DHH), jnp.float32),
            jax.ShapeDtypeStruct((N, 1), jnp.float32),
        ],
    )(deg, x, w1)


def _mid(plo, phi, glo, ghi, dinv, b, w):
    return pl.pallas_call(
        _mid_body,
        grid=(GRID,),
        in_specs=[
            _ROW_SPEC, _ROW_SPEC, _ROW_SPEC, _ROW_SPEC, _DINV_SPEC,
            _VEC_SPEC,
            pl.BlockSpec((D_H, D_H), lambda i: (0, 0)),
        ],
        out_specs=[_ROW_SPEC, _ROW_SPEC],
        out_shape=[
            jax.ShapeDtypeStruct((N, DHH), jnp.float32),
            jax.ShapeDtypeStruct((N, DHH), jnp.float32),
        ],
    )(plo, phi, glo, ghi, dinv, b, w)


def _fin(plo, phi, glo, ghi, dinv, b, batch_r, gamma, beta):
    return pl.pallas_call(
        _fin_body,
        grid=(GRID,),
        in_specs=[
            _ROW_SPEC, _ROW_SPEC, _ROW_SPEC, _ROW_SPEC, _DINV_SPEC,
            _VEC_SPEC,
            pl.BlockSpec((1, 1, BLK), lambda i: (i, 0, 0)),
            _VEC_SPEC, _VEC_SPEC,
        ],
        out_specs=pl.BlockSpec((G_GRAPHS, D_H), lambda i: (0, 0)),
        out_shape=jax.ShapeDtypeStruct((G_GRAPHS, D_H), jnp.float32),
        scratch_shapes=[
            pltpu.VMEM((G_GRAPHS, D_H), jnp.float32),
            pltpu.VMEM((G_GRAPHS, 1), jnp.float32),
        ],
    )(plo, phi, glo, ghi, dinv, b, batch_r, gamma, beta)


def kernel(x, edge_index, edge_weight, batch, W1, b1, W2, b2, W3, b3,
           gamma, beta):
    src = edge_index[0]
    dst = edge_index[1]
    ones128 = jnp.ones((CHUNK, DHH), jnp.float32)
    zeros128 = jnp.zeros((ROWS, DHH), jnp.float32)

    deg = _deg_sc(dst, ones128, zeros128)
    glo1, ghi1, dinv = _mm1(deg, x, W1)
    plo1, phi1 = _prop_sc(glo1, ghi1, src, dst, zeros128)
    glo2, ghi2 = _mid(plo1, phi1, glo1, ghi1, dinv, b1.reshape(1, D_H), W2)
    plo2, phi2 = _prop_sc(glo2, ghi2, src, dst, zeros128)
    glo3, ghi3 = _mid(plo2, phi2, glo2, ghi2, dinv, b2.reshape(1, D_H), W3)
    plo3, phi3 = _prop_sc(glo3, ghi3, src, dst, zeros128)
    return _fin(plo3, phi3, glo3, ghi3, dinv, b3.reshape(1, D_H),
                batch.reshape(GRID, 1, BLK),
                gamma.reshape(1, D_H), beta.reshape(1, D_H))


# speedup vs baseline: 12.3384x; 1.4125x over previous
import functools

import jax
import jax.numpy as jnp
from jax import lax
from jax.experimental import pallas as pl
from jax.experimental.pallas import tpu as pltpu
from jax.experimental.pallas import tpu_sc as plsc

N = 10000
E = 320000
D_IN = 128
D_H = 256
DHH = 128
G_GRAPHS = 64

ROWS = 632
ACC_ROWS = 16 * ROWS
CHUNK = 80
TILE_EDGES = E // 16
TILE_CHUNKS = TILE_EDGES // CHUNK
DEG_TILE_EDGES = E // 32
DEG_TILE_CHUNKS = DEG_TILE_EDGES // CHUNK

BLK = 1000
GRID = N // BLK

_mesh = plsc.VectorSubcoreMesh(core_axis_name="c", subcore_axis_name="s")


def _zero_acc(zeros_hbm, acc, s):
    pltpu.sync_copy(zeros_hbm, acc.at[pl.ds(s * ROWS, ROWS)])


def _write_out(acc, out_hbm, s):
    @pl.when(s < 15)
    def _():
        pltpu.sync_copy(acc.at[pl.ds(s * ROWS, ROWS)],
                        out_hbm.at[pl.ds(s * ROWS, ROWS)])

    @pl.when(s == 15)
    def _():
        rem = N - 15 * ROWS
        pltpu.sync_copy(acc.at[pl.ds(15 * ROWS, rem)],
                        out_hbm.at[pl.ds(15 * ROWS, rem)])


@functools.partial(
    pl.kernel,
    mesh=_mesh,
    out_type=(jax.ShapeDtypeStruct((N, DHH), jnp.float32),
              jax.ShapeDtypeStruct((N, DHH), jnp.float32)),
    scratch_types=[
        pltpu.VMEM((CHUNK,), jnp.int32),
        pltpu.VMEM((CHUNK, DHH), jnp.float32),
        pltpu.VMEM_SHARED((ACC_ROWS, DHH), jnp.float32),
    ],
)
def _deg_sc(dst_hbm, ones_hbm, zeros_hbm, deg0_hbm, deg1_hbm,
            dst_v, ones_v, dacc):
    c = lax.axis_index("c")
    s = lax.axis_index("s")
    _zero_acc(zeros_hbm, dacc, s)
    pltpu.sync_copy(ones_hbm, ones_v)
    plsc.subcore_barrier()
    base0 = (c * 16 + s) * DEG_TILE_EDGES

    def body(i, carry):
        pltpu.sync_copy(dst_hbm.at[pl.ds(base0 + i * CHUNK, CHUNK)], dst_v)
        pltpu.sync_copy(ones_v, dacc.at[dst_v], add=True)
        return carry

    lax.fori_loop(0, DEG_TILE_CHUNKS, body, 0)
    plsc.subcore_barrier()

    @pl.when(c == 0)
    def _():
        _write_out(dacc, deg0_hbm, s)

    @pl.when(c == 1)
    def _():
        _write_out(dacc, deg1_hbm, s)


@functools.partial(
    pl.kernel,
    mesh=_mesh,
    out_type=(jax.ShapeDtypeStruct((N, DHH), jnp.float32),
              jax.ShapeDtypeStruct((N, DHH), jnp.float32)),
    scratch_types=[
        pltpu.VMEM((CHUNK,), jnp.int32),
        pltpu.VMEM((CHUNK,), jnp.int32),
        pltpu.VMEM((CHUNK,), jnp.int32),
        pltpu.VMEM((CHUNK,), jnp.int32),
        pltpu.VMEM((CHUNK, DHH), jnp.float32),
        pltpu.VMEM((CHUNK, DHH), jnp.float32),
        pltpu.VMEM_SHARED((ACC_ROWS, DHH), jnp.float32),
        pltpu.SemaphoreType.DMA,
        pltpu.SemaphoreType.DMA,
    ],
)
def _prop_sc(glo_hbm, ghi_hbm, src_hbm, dst_hbm, zeros_hbm, plo_hbm, phi_hbm,
             src0, dst0, src1, dst1, buf0, buf1, acc, sem0, sem1):
    c = lax.axis_index("c")
    s = lax.axis_index("s")
    _zero_acc(zeros_hbm, acc, s)
    plsc.subcore_barrier()
    base0 = s * TILE_EDGES

    def pipeline(gref):
        def stage(chunk_i, src_v, dst_v, buf, sem):
            base = base0 + chunk_i * CHUNK
            pltpu.sync_copy(dst_hbm.at[pl.ds(base, CHUNK)], dst_v)
            pltpu.sync_copy(src_hbm.at[pl.ds(base, CHUNK)], src_v)
            pltpu.async_copy(gref.at[src_v], buf, sem)

        def consume(src_v, dst_v, buf, sem):
            pltpu.make_async_copy(gref.at[src_v], buf, sem).wait()
            pltpu.sync_copy(buf, acc.at[dst_v], add=True)

        stage(0, src0, dst0, buf0, sem0)

        def body(j, carry):
            stage(2 * j + 1, src1, dst1, buf1, sem1)
            consume(src0, dst0, buf0, sem0)

            @pl.when(j < TILE_CHUNKS // 2 - 1)
            def _():
                stage(2 * j + 2, src0, dst0, buf0, sem0)

            consume(src1, dst1, buf1, sem1)
            return carry

        lax.fori_loop(0, TILE_CHUNKS // 2, body, 0)

    @pl.when(c == 0)
    def _():
        pipeline(glo_hbm)

    @pl.when(c == 1)
    def _():
        pipeline(ghi_hbm)

    plsc.subcore_barrier()

    @pl.when(c == 0)
    def _():
        _write_out(acc, plo_hbm, s)

    @pl.when(c == 1)
    def _():
        _write_out(acc, phi_hbm, s)


def _mm1_body(d0_ref, d1_ref, x_ref, w_ref, glo_ref, ghi_ref, dinv_ref):
    deg = d0_ref[:, :1] + d1_ref[:, :1] + 1.0
    dinv = lax.rsqrt(deg)
    g = dinv * jnp.dot(x_ref[...], w_ref[...],
                       preferred_element_type=jnp.float32)
    glo_ref[...] = g[:, :DHH]
    ghi_ref[...] = g[:, DHH:]
    dinv_ref[...] = dinv


def _mid_body(plo_ref, phi_ref, glo_ref, ghi_ref, dinv_ref, b_ref, w_ref,
              olo_ref, ohi_ref):
    dinv = dinv_ref[...]
    pg = jnp.concatenate([plo_ref[...] + glo_ref[...],
                          phi_ref[...] + ghi_ref[...]], axis=1)
    h = jnp.maximum(dinv * pg + b_ref[...], 0.0)
    g = dinv * jnp.dot(h, w_ref[...], preferred_element_type=jnp.float32)
    olo_ref[...] = g[:, :DHH]
    ohi_ref[...] = g[:, DHH:]


def _fin_body(plo_ref, phi_ref, glo_ref, ghi_ref, dinv_ref, b_ref, batch_ref,
              gamma_ref, beta_ref, out_ref, pool_acc, cnt_acc):
    i = pl.program_id(0)

    @pl.when(i == 0)
    def _():
        pool_acc[...] = jnp.zeros_like(pool_acc)
        cnt_acc[...] = jnp.zeros_like(cnt_acc)

    dinv = dinv_ref[...]
    pg = jnp.concatenate([plo_ref[...] + glo_ref[...],
                          phi_ref[...] + ghi_ref[...]], axis=1)
    h = jnp.maximum(dinv * pg + b_ref[...], 0.0)
    ids = lax.broadcasted_iota(jnp.int32, (G_GRAPHS, BLK), 0)
    onehot = (ids == batch_ref[0]).astype(jnp.float32)
    pool_acc[...] += jnp.dot(onehot, h, preferred_element_type=jnp.float32)
    cnt_acc[...] += jnp.sum(onehot, axis=1, keepdims=True)

    @pl.when(i == GRID - 1)
    def _():
        pooled = pool_acc[...] / jnp.maximum(cnt_acc[...], 1.0)
        mu = jnp.mean(pooled, axis=-1, keepdims=True)
        var = jnp.mean((pooled - mu) ** 2, axis=-1, keepdims=True)
        out_ref[...] = ((pooled - mu) * lax.rsqrt(var + 1e-5)
                        * gamma_ref[...] + beta_ref[...])


_ROW_SPEC = pl.BlockSpec((BLK, DHH), lambda i: (i, 0))
_DINV_SPEC = pl.BlockSpec((BLK, 1), lambda i: (i, 0))
_VEC_SPEC = pl.BlockSpec((1, D_H), lambda i: (0, 0))


def _mm1(d0, d1, x, w1):
    return pl.pallas_call(
        _mm1_body,
        grid=(GRID,),
        in_specs=[
            _ROW_SPEC,
            _ROW_SPEC,
            pl.BlockSpec((BLK, D_IN), lambda i: (i, 0)),
            pl.BlockSpec((D_IN, D_H), lambda i: (0, 0)),
        ],
        out_specs=[_ROW_SPEC, _ROW_SPEC, _DINV_SPEC],
        out_shape=[
            jax.ShapeDtypeStruct((N, DHH), jnp.float32),
            jax.ShapeDtypeStruct((N, DHH), jnp.float32),
            jax.ShapeDtypeStruct((N, 1), jnp.float32),
        ],
    )(d0, d1, x, w1)


def _mid(plo, phi, glo, ghi, dinv, b, w):
    return pl.pallas_call(
        _mid_body,
        grid=(GRID,),
        in_specs=[
            _ROW_SPEC, _ROW_SPEC, _ROW_SPEC, _ROW_SPEC, _DINV_SPEC,
            _VEC_SPEC,
            pl.BlockSpec((D_H, D_H), lambda i: (0, 0)),
        ],
        out_specs=[_ROW_SPEC, _ROW_SPEC],
        out_shape=[
            jax.ShapeDtypeStruct((N, DHH), jnp.float32),
            jax.ShapeDtypeStruct((N, DHH), jnp.float32),
        ],
    )(plo, phi, glo, ghi, dinv, b, w)


def _fin(plo, phi, glo, ghi, dinv, b, batch_r, gamma, beta):
    return pl.pallas_call(
        _fin_body,
        grid=(GRID,),
        in_specs=[
            _ROW_SPEC, _ROW_SPEC, _ROW_SPEC, _ROW_SPEC, _DINV_SPEC,
            _VEC_SPEC,
            pl.BlockSpec((1, 1, BLK), lambda i: (i, 0, 0)),
            _VEC_SPEC, _VEC_SPEC,
        ],
        out_specs=pl.BlockSpec((G_GRAPHS, D_H), lambda i: (0, 0)),
        out_shape=jax.ShapeDtypeStruct((G_GRAPHS, D_H), jnp.float32),
        scratch_shapes=[
            pltpu.VMEM((G_GRAPHS, D_H), jnp.float32),
            pltpu.VMEM((G_GRAPHS, 1), jnp.float32),
        ],
    )(plo, phi, glo, ghi, dinv, b, batch_r, gamma, beta)


def kernel(x, edge_index, edge_weight, batch, W1, b1, W2, b2, W3, b3,
           gamma, beta):
    src = edge_index[0]
    dst = edge_index[1]
    ones128 = jnp.ones((CHUNK, DHH), jnp.float32)
    zeros128 = jnp.zeros((ROWS, DHH), jnp.float32)

    d0, d1 = _deg_sc(dst, ones128, zeros128)
    glo1, ghi1, dinv = _mm1(d0, d1, x, W1)
    plo1, phi1 = _prop_sc(glo1, ghi1, src, dst, zeros128)
    glo2, ghi2 = _mid(plo1, phi1, glo1, ghi1, dinv, b1.reshape(1, D_H), W2)
    plo2, phi2 = _prop_sc(glo2, ghi2, src, dst, zeros128)
    glo3, ghi3 = _mid(plo2, phi2, glo2, ghi2, dinv, b2.reshape(1, D_H), W3)
    plo3, phi3 = _prop_sc(glo3, ghi3, src, dst, zeros128)
    return _fin(plo3, phi3, glo3, ghi3, dinv, b3.reshape(1, D_H),
                batch.reshape(GRID, 1, BLK),
                gamma.reshape(1, D_H), beta.reshape(1, D_H))


# speedup vs baseline: 17.2781x; 1.4004x over previous
import functools

import jax
import jax.numpy as jnp
from jax import lax
from jax.experimental import pallas as pl
from jax.experimental.pallas import tpu as pltpu
from jax.experimental.pallas import tpu_sc as plsc

N = 10000
E = 320000
D_IN = 128
D_H = 256
DHH = 128
G_GRAPHS = 64

ROWS = 632
ACC_ROWS = 16 * ROWS
CHUNK = 80
TILE_EDGES = E // 16
TILE_CHUNKS = TILE_EDGES // CHUNK
DEG_TILE_EDGES = E // 32
DEG_TILE_CHUNKS = DEG_TILE_EDGES // CHUNK

BLK = 1000
GRID = N // BLK

_mesh = plsc.VectorSubcoreMesh(core_axis_name="c", subcore_axis_name="s")


def _zero_acc(zeros_hbm, acc, s):
    pltpu.sync_copy(zeros_hbm, acc.at[pl.ds(s * ROWS, ROWS)])


def _write_out(acc, out_hbm, s):
    @pl.when(s < 15)
    def _():
        pltpu.sync_copy(acc.at[pl.ds(s * ROWS, ROWS)],
                        out_hbm.at[pl.ds(s * ROWS, ROWS)])

    @pl.when(s == 15)
    def _():
        rem = N - 15 * ROWS
        pltpu.sync_copy(acc.at[pl.ds(15 * ROWS, rem)],
                        out_hbm.at[pl.ds(15 * ROWS, rem)])


@functools.partial(
    pl.kernel,
    mesh=_mesh,
    out_type=(jax.ShapeDtypeStruct((N, DHH), jnp.float32),
              jax.ShapeDtypeStruct((N, DHH), jnp.float32)),
    scratch_types=[
        pltpu.VMEM((CHUNK,), jnp.int32),
        pltpu.VMEM((CHUNK, DHH), jnp.float32),
        pltpu.VMEM_SHARED((ACC_ROWS, DHH), jnp.float32),
    ],
)
def _deg_sc(dst_hbm, ones_hbm, zeros_hbm, deg0_hbm, deg1_hbm,
            dst_v, ones_v, dacc):
    c = lax.axis_index("c")
    s = lax.axis_index("s")
    _zero_acc(zeros_hbm, dacc, s)
    pltpu.sync_copy(ones_hbm, ones_v)
    plsc.subcore_barrier()
    base0 = (c * 16 + s) * DEG_TILE_EDGES

    def body(i, carry):
        pltpu.sync_copy(dst_hbm.at[pl.ds(base0 + i * CHUNK, CHUNK)], dst_v)
        pltpu.sync_copy(ones_v, dacc.at[dst_v], add=True)
        return carry

    lax.fori_loop(0, DEG_TILE_CHUNKS, body, 0)
    plsc.subcore_barrier()

    @pl.when(c == 0)
    def _():
        _write_out(dacc, deg0_hbm, s)

    @pl.when(c == 1)
    def _():
        _write_out(dacc, deg1_hbm, s)


SB = 25
NSB = TILE_CHUNKS // SB
SB_EDGES = SB * CHUNK


@functools.partial(
    pl.kernel,
    mesh=_mesh,
    out_type=(jax.ShapeDtypeStruct((N, DHH), jnp.float32),
              jax.ShapeDtypeStruct((N, DHH), jnp.float32)),
    scratch_types=[
        pltpu.VMEM((SB_EDGES,), jnp.int32),
        pltpu.VMEM((SB_EDGES,), jnp.int32),
        pltpu.VMEM((SB_EDGES,), jnp.int32),
        pltpu.VMEM((SB_EDGES,), jnp.int32),
        pltpu.VMEM((CHUNK,), jnp.int32),
        pltpu.VMEM((CHUNK,), jnp.int32),
        pltpu.VMEM((CHUNK, DHH), jnp.float32),
        pltpu.VMEM((CHUNK, DHH), jnp.float32),
        pltpu.VMEM_SHARED((ACC_ROWS, DHH), jnp.float32),
        pltpu.SemaphoreType.DMA,
        pltpu.SemaphoreType.DMA,
        pltpu.SemaphoreType.DMA,
        pltpu.SemaphoreType.DMA,
    ],
)
def _prop_sc(glo_hbm, ghi_hbm, src_hbm, dst_hbm, zeros_hbm, plo_hbm, phi_hbm,
             sA, dA, sB, dB, di0, di1, buf0, buf1, acc,
             sem0, sem1, semA, semB):
    c = lax.axis_index("c")
    s = lax.axis_index("s")
    _zero_acc(zeros_hbm, acc, s)
    plsc.subcore_barrier()
    base0 = s * TILE_EDGES

    def pipeline(gref):
        bufs = (buf0, buf1)
        sems = (sem0, sem1)
        dis = (di0, di1)

        def idx_fire(sb_i, s_blk, d_blk, sem):
            base = base0 + sb_i * SB_EDGES
            pltpu.async_copy(src_hbm.at[pl.ds(base, SB_EDGES)], s_blk, sem)
            pltpu.async_copy(dst_hbm.at[pl.ds(base, SB_EDGES)], d_blk, sem)

        def idx_wait(sb_i, s_blk, d_blk, sem):
            base = base0 + sb_i * SB_EDGES
            pltpu.make_async_copy(src_hbm.at[pl.ds(base, SB_EDGES)],
                                  s_blk, sem).wait()
            pltpu.make_async_copy(dst_hbm.at[pl.ds(base, SB_EDGES)],
                                  d_blk, sem).wait()

        def gather_fire(s_blk, j, buf, sem):
            pltpu.async_copy(gref.at[s_blk.at[pl.ds(j * CHUNK, CHUNK)]],
                             buf, sem)

        def consume(s_blk, d_blk, j, buf, sem, d_cur):
            pltpu.make_async_copy(gref.at[s_blk.at[pl.ds(j * CHUNK, CHUNK)]],
                                  buf, sem).wait()
            pltpu.sync_copy(buf, acc.at[d_blk.at[pl.ds(j * CHUNK, CHUNK)]],
                            add=True)

        def superblock(sb_i, s_blk, d_blk):
            for j in range(SB):
                if j + 1 < SB:
                    gather_fire(s_blk, j + 1, bufs[(j + 1) % 2],
                                sems[(j + 1) % 2])
                consume(s_blk, d_blk, j, bufs[j % 2], sems[j % 2],
                        dis[j % 2])

        idx_fire(0, sA, dA, semA)
        idx_wait(0, sA, dA, semA)
        gather_fire(sA, 0, buf0, sem0)

        def body(k2, carry):
            a = 2 * k2
            idx_fire(a + 1, sB, dB, semB)
            superblock(a, sA, dA)
            idx_wait(a + 1, sB, dB, semB)
            gather_fire(sB, 0, buf0, sem0)

            @pl.when(k2 < NSB // 2 - 1)
            def _():
                idx_fire(a + 2, sA, dA, semA)

            superblock(a + 1, sB, dB)

            @pl.when(k2 < NSB // 2 - 1)
            def _():
                idx_wait(a + 2, sA, dA, semA)
                gather_fire(sA, 0, buf0, sem0)

            return carry

        lax.fori_loop(0, NSB // 2, body, 0)

    @pl.when(c == 0)
    def _():
        pipeline(glo_hbm)

    @pl.when(c == 1)
    def _():
        pipeline(ghi_hbm)

    plsc.subcore_barrier()

    @pl.when(c == 0)
    def _():
        _write_out(acc, plo_hbm, s)

    @pl.when(c == 1)
    def _():
        _write_out(acc, phi_hbm, s)


def _mm1_body(d0_ref, d1_ref, x_ref, w_ref, glo_ref, ghi_ref, dinv_ref):
    deg = d0_ref[:, :1] + d1_ref[:, :1] + 1.0
    dinv = lax.rsqrt(deg)
    g = dinv * jnp.dot(x_ref[...], w_ref[...],
                       preferred_element_type=jnp.float32)
    glo_ref[...] = g[:, :DHH]
    ghi_ref[...] = g[:, DHH:]
    dinv_ref[...] = dinv


def _mid_body(plo_ref, phi_ref, glo_ref, ghi_ref, dinv_ref, b_ref, w_ref,
              olo_ref, ohi_ref):
    dinv = dinv_ref[...]
    pg = jnp.concatenate([plo_ref[...] + glo_ref[...],
                          phi_ref[...] + ghi_ref[...]], axis=1)
    h = jnp.maximum(dinv * pg + b_ref[...], 0.0)
    g = dinv * jnp.dot(h, w_ref[...], preferred_element_type=jnp.float32)
    olo_ref[...] = g[:, :DHH]
    ohi_ref[...] = g[:, DHH:]


def _fin_body(plo_ref, phi_ref, glo_ref, ghi_ref, dinv_ref, b_ref, batch_ref,
              gamma_ref, beta_ref, out_ref, pool_acc, cnt_acc):
    i = pl.program_id(0)

    @pl.when(i == 0)
    def _():
        pool_acc[...] = jnp.zeros_like(pool_acc)
        cnt_acc[...] = jnp.zeros_like(cnt_acc)

    dinv = dinv_ref[...]
    pg = jnp.concatenate([plo_ref[...] + glo_ref[...],
                          phi_ref[...] + ghi_ref[...]], axis=1)
    h = jnp.maximum(dinv * pg + b_ref[...], 0.0)
    ids = lax.broadcasted_iota(jnp.int32, (G_GRAPHS, BLK), 0)
    onehot = (ids == batch_ref[0]).astype(jnp.float32)
    pool_acc[...] += jnp.dot(onehot, h, preferred_element_type=jnp.float32)
    cnt_acc[...] += jnp.sum(onehot, axis=1, keepdims=True)

    @pl.when(i == GRID - 1)
    def _():
        pooled = pool_acc[...] / jnp.maximum(cnt_acc[...], 1.0)
        mu = jnp.mean(pooled, axis=-1, keepdims=True)
        var = jnp.mean((pooled - mu) ** 2, axis=-1, keepdims=True)
        out_ref[...] = ((pooled - mu) * lax.rsqrt(var + 1e-5)
                        * gamma_ref[...] + beta_ref[...])


_ROW_SPEC = pl.BlockSpec((BLK, DHH), lambda i: (i, 0))
_DINV_SPEC = pl.BlockSpec((BLK, 1), lambda i: (i, 0))
_VEC_SPEC = pl.BlockSpec((1, D_H), lambda i: (0, 0))


def _mm1(d0, d1, x, w1):
    return pl.pallas_call(
        _mm1_body,
        grid=(GRID,),
        in_specs=[
            _ROW_SPEC,
            _ROW_SPEC,
            pl.BlockSpec((BLK, D_IN), lambda i: (i, 0)),
            pl.BlockSpec((D_IN, D_H), lambda i: (0, 0)),
        ],
        out_specs=[_ROW_SPEC, _ROW_SPEC, _DINV_SPEC],
        out_shape=[
            jax.ShapeDtypeStruct((N, DHH), jnp.float32),
            jax.ShapeDtypeStruct((N, DHH), jnp.float32),
            jax.ShapeDtypeStruct((N, 1), jnp.float32),
        ],
    )(d0, d1, x, w1)


def _mid(plo, phi, glo, ghi, dinv, b, w):
    return pl.pallas_call(
        _mid_body,
        grid=(GRID,),
        in_specs=[
            _ROW_SPEC, _ROW_SPEC, _ROW_SPEC, _ROW_SPEC, _DINV_SPEC,
            _VEC_SPEC,
            pl.BlockSpec((D_H, D_H), lambda i: (0, 0)),
        ],
        out_specs=[_ROW_SPEC, _ROW_SPEC],
        out_shape=[
            jax.ShapeDtypeStruct((N, DHH), jnp.float32),
            jax.ShapeDtypeStruct((N, DHH), jnp.float32),
        ],
    )(plo, phi, glo, ghi, dinv, b, w)


def _fin(plo, phi, glo, ghi, dinv, b, batch_r, gamma, beta):
    return pl.pallas_call(
        _fin_body,
        grid=(GRID,),
        in_specs=[
            _ROW_SPEC, _ROW_SPEC, _ROW_SPEC, _ROW_SPEC, _DINV_SPEC,
            _VEC_SPEC,
            pl.BlockSpec((1, 1, BLK), lambda i: (i, 0, 0)),
            _VEC_SPEC, _VEC_SPEC,
        ],
        out_specs=pl.BlockSpec((G_GRAPHS, D_H), lambda i: (0, 0)),
        out_shape=jax.ShapeDtypeStruct((G_GRAPHS, D_H), jnp.float32),
        scratch_shapes=[
            pltpu.VMEM((G_GRAPHS, D_H), jnp.float32),
            pltpu.VMEM((G_GRAPHS, 1), jnp.float32),
        ],
    )(plo, phi, glo, ghi, dinv, b, batch_r, gamma, beta)


def kernel(x, edge_index, edge_weight, batch, W1, b1, W2, b2, W3, b3,
           gamma, beta):
    src = edge_index[0]
    dst = edge_index[1]
    ones128 = jnp.ones((CHUNK, DHH), jnp.float32)
    zeros128 = jnp.zeros((ROWS, DHH), jnp.float32)

    d0, d1 = _deg_sc(dst, ones128, zeros128)
    glo1, ghi1, dinv = _mm1(d0, d1, x, W1)
    plo1, phi1 = _prop_sc(glo1, ghi1, src, dst, zeros128)
    glo2, ghi2 = _mid(plo1, phi1, glo1, ghi1, dinv, b1.reshape(1, D_H), W2)
    plo2, phi2 = _prop_sc(glo2, ghi2, src, dst, zeros128)
    glo3, ghi3 = _mid(plo2, phi2, glo2, ghi2, dinv, b2.reshape(1, D_H), W3)
    plo3, phi3 = _prop_sc(glo3, ghi3, src, dst, zeros128)
    return _fin(plo3, phi3, glo3, ghi3, dinv, b3.reshape(1, D_H),
                batch.reshape(GRID, 1, BLK),
                gamma.reshape(1, D_H), beta.reshape(1, D_H))


# speedup vs baseline: 18.2770x; 1.0578x over previous
import functools

import jax
import jax.numpy as jnp
from jax import lax
from jax.experimental import pallas as pl
from jax.experimental.pallas import tpu as pltpu
from jax.experimental.pallas import tpu_sc as plsc

N = 10000
E = 320000
D_IN = 128
D_H = 256
DHH = 128
G_GRAPHS = 64

ROWS = 632
ACC_ROWS = 16 * ROWS
CHUNK = 80
TILE_EDGES = E // 16
TILE_CHUNKS = TILE_EDGES // CHUNK
DEG_TILE_EDGES = E // 32
DEG_TILE_CHUNKS = DEG_TILE_EDGES // CHUNK

BLK = 1000
GRID = N // BLK

_mesh = plsc.VectorSubcoreMesh(core_axis_name="c", subcore_axis_name="s")


def _zero_acc(zeros_hbm, acc, s):
    pltpu.sync_copy(zeros_hbm, acc.at[pl.ds(s * ROWS, ROWS)])


def _write_out(acc, out_hbm, s):
    @pl.when(s < 15)
    def _():
        pltpu.sync_copy(acc.at[pl.ds(s * ROWS, ROWS)],
                        out_hbm.at[pl.ds(s * ROWS, ROWS)])

    @pl.when(s == 15)
    def _():
        rem = N - 15 * ROWS
        pltpu.sync_copy(acc.at[pl.ds(15 * ROWS, rem)],
                        out_hbm.at[pl.ds(15 * ROWS, rem)])


DEG_SB = 25
DEG_NSB = DEG_TILE_CHUNKS // DEG_SB
DEG_SB_EDGES = DEG_SB * CHUNK


@functools.partial(
    pl.kernel,
    mesh=_mesh,
    out_type=(jax.ShapeDtypeStruct((N, DHH), jnp.float32),
              jax.ShapeDtypeStruct((N, DHH), jnp.float32)),
    scratch_types=[
        pltpu.VMEM((DEG_SB_EDGES,), jnp.int32),
        pltpu.VMEM((CHUNK, DHH), jnp.float32),
        pltpu.VMEM_SHARED((ACC_ROWS, DHH), jnp.float32),
        pltpu.SemaphoreType.DMA,
        pltpu.SemaphoreType.DMA,
    ],
)
def _deg_sc(dst_hbm, ones_hbm, zeros_hbm, deg0_hbm, deg1_hbm,
            d_blk, ones_v, dacc, ssem0, ssem1):
    c = lax.axis_index("c")
    s = lax.axis_index("s")
    _zero_acc(zeros_hbm, dacc, s)
    pltpu.sync_copy(ones_hbm, ones_v)
    plsc.subcore_barrier()
    base0 = (c * 16 + s) * DEG_TILE_EDGES
    ssems = (ssem0, ssem1)

    def scat_wait(ssem):
        pltpu.make_async_copy(ones_v, dacc.at[d_blk.at[pl.ds(0, CHUNK)]],
                              ssem).wait()

    def body(k, carry):
        pltpu.sync_copy(dst_hbm.at[pl.ds(base0 + k * DEG_SB_EDGES,
                                         DEG_SB_EDGES)], d_blk)
        for j in range(DEG_SB):
            if j >= 2:
                scat_wait(ssems[j % 2])
            pltpu.async_copy(ones_v,
                             dacc.at[d_blk.at[pl.ds(j * CHUNK, CHUNK)]],
                             ssems[j % 2], add=True)
        scat_wait(ssems[(DEG_SB - 2) % 2])
        scat_wait(ssems[(DEG_SB - 1) % 2])
        return carry

    lax.fori_loop(0, DEG_NSB, body, 0)
    plsc.subcore_barrier()

    @pl.when(c == 0)
    def _():
        _write_out(dacc, deg0_hbm, s)

    @pl.when(c == 1)
    def _():
        _write_out(dacc, deg1_hbm, s)


SB = 25
NSB = TILE_CHUNKS // SB
SB_EDGES = SB * CHUNK


@functools.partial(
    pl.kernel,
    mesh=_mesh,
    out_type=(jax.ShapeDtypeStruct((N, DHH), jnp.float32),
              jax.ShapeDtypeStruct((N, DHH), jnp.float32)),
    scratch_types=[
        pltpu.VMEM((SB_EDGES,), jnp.int32),
        pltpu.VMEM((SB_EDGES,), jnp.int32),
        pltpu.VMEM((SB_EDGES,), jnp.int32),
        pltpu.VMEM((SB_EDGES,), jnp.int32),
        pltpu.VMEM((CHUNK, DHH), jnp.float32),
        pltpu.VMEM((CHUNK, DHH), jnp.float32),
        pltpu.VMEM_SHARED((ACC_ROWS, DHH), jnp.float32),
        pltpu.SemaphoreType.DMA,
        pltpu.SemaphoreType.DMA,
        pltpu.SemaphoreType.DMA,
        pltpu.SemaphoreType.DMA,
        pltpu.SemaphoreType.DMA,
        pltpu.SemaphoreType.DMA,
    ],
)
def _prop_sc(glo_hbm, ghi_hbm, src_hbm, dst_hbm, zeros_hbm, plo_hbm, phi_hbm,
             sA, dA, sB, dB, buf0, buf1, acc,
             sem0, sem1, semA, semB, ssem0, ssem1):
    c = lax.axis_index("c")
    s = lax.axis_index("s")
    _zero_acc(zeros_hbm, acc, s)
    plsc.subcore_barrier()
    base0 = s * TILE_EDGES

    def pipeline(gref):
        bufs = (buf0, buf1)
        sems = (sem0, sem1)
        ssems = (ssem0, ssem1)

        def idx_fire(sb_i, s_blk, d_blk, sem):
            base = base0 + sb_i * SB_EDGES
            pltpu.async_copy(src_hbm.at[pl.ds(base, SB_EDGES)], s_blk, sem)
            pltpu.async_copy(dst_hbm.at[pl.ds(base, SB_EDGES)], d_blk, sem)

        def idx_wait(sb_i, s_blk, d_blk, sem):
            base = base0 + sb_i * SB_EDGES
            pltpu.make_async_copy(src_hbm.at[pl.ds(base, SB_EDGES)],
                                  s_blk, sem).wait()
            pltpu.make_async_copy(dst_hbm.at[pl.ds(base, SB_EDGES)],
                                  d_blk, sem).wait()

        def gather_fire(s_blk, j, buf, sem):
            pltpu.async_copy(gref.at[s_blk.at[pl.ds(j * CHUNK, CHUNK)]],
                             buf, sem)

        def scat_wait(d_blk, buf, ssem):
            pltpu.make_async_copy(buf, acc.at[d_blk.at[pl.ds(0, CHUNK)]],
                                  ssem).wait()

        def consume(s_blk, d_blk, j, buf, sem, ssem):
            pltpu.make_async_copy(gref.at[s_blk.at[pl.ds(j * CHUNK, CHUNK)]],
                                  buf, sem).wait()
            pltpu.async_copy(buf, acc.at[d_blk.at[pl.ds(j * CHUNK, CHUNK)]],
                             ssem, add=True)

        def superblock(sb_i, s_blk, d_blk):
            for j in range(SB):
                if j + 1 < SB:
                    if j >= 1:
                        scat_wait(d_blk, bufs[(j + 1) % 2], ssems[(j + 1) % 2])
                    gather_fire(s_blk, j + 1, bufs[(j + 1) % 2],
                                sems[(j + 1) % 2])
                consume(s_blk, d_blk, j, bufs[j % 2], sems[j % 2],
                        ssems[j % 2])
            scat_wait(d_blk, bufs[(SB - 1) % 2], ssems[(SB - 1) % 2])
            scat_wait(d_blk, bufs[SB % 2], ssems[SB % 2])

        idx_fire(0, sA, dA, semA)
        idx_wait(0, sA, dA, semA)
        gather_fire(sA, 0, buf0, sem0)

        def body(k2, carry):
            a = 2 * k2
            idx_fire(a + 1, sB, dB, semB)
            superblock(a, sA, dA)
            idx_wait(a + 1, sB, dB, semB)
            gather_fire(sB, 0, buf0, sem0)

            @pl.when(k2 < NSB // 2 - 1)
            def _():
                idx_fire(a + 2, sA, dA, semA)

            superblock(a + 1, sB, dB)

            @pl.when(k2 < NSB // 2 - 1)
            def _():
                idx_wait(a + 2, sA, dA, semA)
                gather_fire(sA, 0, buf0, sem0)

            return carry

        lax.fori_loop(0, NSB // 2, body, 0)

    @pl.when(c == 0)
    def _():
        pipeline(glo_hbm)

    @pl.when(c == 1)
    def _():
        pipeline(ghi_hbm)

    plsc.subcore_barrier()

    @pl.when(c == 0)
    def _():
        _write_out(acc, plo_hbm, s)

    @pl.when(c == 1)
    def _():
        _write_out(acc, phi_hbm, s)


def _mm1_body(d0_ref, d1_ref, x_ref, w_ref, glo_ref, ghi_ref, dinv_ref):
    deg = d0_ref[:, :1] + d1_ref[:, :1] + 1.0
    dinv = lax.rsqrt(deg)
    g = dinv * jnp.dot(x_ref[...], w_ref[...],
                       preferred_element_type=jnp.float32)
    glo_ref[...] = g[:, :DHH]
    ghi_ref[...] = g[:, DHH:]
    dinv_ref[...] = dinv


def _mid_body(plo_ref, phi_ref, glo_ref, ghi_ref, dinv_ref, b_ref, w_ref,
              olo_ref, ohi_ref):
    dinv = dinv_ref[...]
    pg = jnp.concatenate([plo_ref[...] + glo_ref[...],
                          phi_ref[...] + ghi_ref[...]], axis=1)
    h = jnp.maximum(dinv * pg + b_ref[...], 0.0)
    g = dinv * jnp.dot(h, w_ref[...], preferred_element_type=jnp.float32)
    olo_ref[...] = g[:, :DHH]
    ohi_ref[...] = g[:, DHH:]


def _fin_body(plo_ref, phi_ref, glo_ref, ghi_ref, dinv_ref, b_ref, batch_ref,
              gamma_ref, beta_ref, out_ref, pool_acc, cnt_acc):
    i = pl.program_id(0)

    @pl.when(i == 0)
    def _():
        pool_acc[...] = jnp.zeros_like(pool_acc)
        cnt_acc[...] = jnp.zeros_like(cnt_acc)

    dinv = dinv_ref[...]
    pg = jnp.concatenate([plo_ref[...] + glo_ref[...],
                          phi_ref[...] + ghi_ref[...]], axis=1)
    h = jnp.maximum(dinv * pg + b_ref[...], 0.0)
    ids = lax.broadcasted_iota(jnp.int32, (G_GRAPHS, BLK), 0)
    onehot = (ids == batch_ref[0]).astype(jnp.float32)
    pool_acc[...] += jnp.dot(onehot, h, preferred_element_type=jnp.float32)
    cnt_acc[...] += jnp.sum(onehot, axis=1, keepdims=True)

    @pl.when(i == GRID - 1)
    def _():
        pooled = pool_acc[...] / jnp.maximum(cnt_acc[...], 1.0)
        mu = jnp.mean(pooled, axis=-1, keepdims=True)
        var = jnp.mean((pooled - mu) ** 2, axis=-1, keepdims=True)
        out_ref[...] = ((pooled - mu) * lax.rsqrt(var + 1e-5)
                        * gamma_ref[...] + beta_ref[...])


_ROW_SPEC = pl.BlockSpec((BLK, DHH), lambda i: (i, 0))
_DINV_SPEC = pl.BlockSpec((BLK, 1), lambda i: (i, 0))
_VEC_SPEC = pl.BlockSpec((1, D_H), lambda i: (0, 0))


def _mm1(d0, d1, x, w1):
    return pl.pallas_call(
        _mm1_body,
        grid=(GRID,),
        in_specs=[
            _ROW_SPEC,
            _ROW_SPEC,
            pl.BlockSpec((BLK, D_IN), lambda i: (i, 0)),
            pl.BlockSpec((D_IN, D_H), lambda i: (0, 0)),
        ],
        out_specs=[_ROW_SPEC, _ROW_SPEC, _DINV_SPEC],
        out_shape=[
            jax.ShapeDtypeStruct((N, DHH), jnp.float32),
            jax.ShapeDtypeStruct((N, DHH), jnp.float32),
            jax.ShapeDtypeStruct((N, 1), jnp.float32),
        ],
    )(d0, d1, x, w1)


def _mid(plo, phi, glo, ghi, dinv, b, w):
    return pl.pallas_call(
        _mid_body,
        grid=(GRID,),
        in_specs=[
            _ROW_SPEC, _ROW_SPEC, _ROW_SPEC, _ROW_SPEC, _DINV_SPEC,
            _VEC_SPEC,
            pl.BlockSpec((D_H, D_H), lambda i: (0, 0)),
        ],
        out_specs=[_ROW_SPEC, _ROW_SPEC],
        out_shape=[
            jax.ShapeDtypeStruct((N, DHH), jnp.float32),
            jax.ShapeDtypeStruct((N, DHH), jnp.float32),
        ],
    )(plo, phi, glo, ghi, dinv, b, w)


def _fin(plo, phi, glo, ghi, dinv, b, batch_r, gamma, beta):
    return pl.pallas_call(
        _fin_body,
        grid=(GRID,),
        in_specs=[
            _ROW_SPEC, _ROW_SPEC, _ROW_SPEC, _ROW_SPEC, _DINV_SPEC,
            _VEC_SPEC,
            pl.BlockSpec((1, 1, BLK), lambda i: (i, 0, 0)),
            _VEC_SPEC, _VEC_SPEC,
        ],
        out_specs=pl.BlockSpec((G_GRAPHS, D_H), lambda i: (0, 0)),
        out_shape=jax.ShapeDtypeStruct((G_GRAPHS, D_H), jnp.float32),
        scratch_shapes=[
            pltpu.VMEM((G_GRAPHS, D_H), jnp.float32),
            pltpu.VMEM((G_GRAPHS, 1), jnp.float32),
        ],
    )(plo, phi, glo, ghi, dinv, b, batch_r, gamma, beta)


def kernel(x, edge_index, edge_weight, batch, W1, b1, W2, b2, W3, b3,
           gamma, beta):
    src = edge_index[0]
    dst = edge_index[1]
    ones128 = jnp.ones((CHUNK, DHH), jnp.float32)
    zeros128 = jnp.zeros((ROWS, DHH), jnp.float32)

    d0, d1 = _deg_sc(dst, ones128, zeros128)
    glo1, ghi1, dinv = _mm1(d0, d1, x, W1)
    plo1, phi1 = _prop_sc(glo1, ghi1, src, dst, zeros128)
    glo2, ghi2 = _mid(plo1, phi1, glo1, ghi1, dinv, b1.reshape(1, D_H), W2)
    plo2, phi2 = _prop_sc(glo2, ghi2, src, dst, zeros128)
    glo3, ghi3 = _mid(plo2, phi2, glo2, ghi2, dinv, b2.reshape(1, D_H), W3)
    plo3, phi3 = _prop_sc(glo3, ghi3, src, dst, zeros128)
    return _fin(plo3, phi3, glo3, ghi3, dinv, b3.reshape(1, D_H),
                batch.reshape(GRID, 1, BLK),
                gamma.reshape(1, D_H), beta.reshape(1, D_H))


# speedup vs baseline: 21.2918x; 1.1650x over previous
import functools

import jax
import jax.numpy as jnp
from jax import lax
from jax.experimental import pallas as pl
from jax.experimental.pallas import tpu as pltpu
from jax.experimental.pallas import tpu_sc as plsc

N = 10000
E = 320000
D_IN = 128
D_H = 256
DHH = 128
G_GRAPHS = 64

ROWS = 632
ACC_ROWS = 16 * ROWS
CHUNK = 80
TILE_EDGES = E // 16
TILE_CHUNKS = TILE_EDGES // CHUNK
DEG_TILE_EDGES = E // 32
DEG_TILE_CHUNKS = DEG_TILE_EDGES // CHUNK

BLK = 1000
GRID = N // BLK

_mesh = plsc.VectorSubcoreMesh(core_axis_name="c", subcore_axis_name="s")


def _zero_acc(zeros_hbm, acc, s):
    pltpu.sync_copy(zeros_hbm, acc.at[pl.ds(s * ROWS, ROWS)])


def _write_out(acc, out_hbm, s):
    @pl.when(s < 15)
    def _():
        pltpu.sync_copy(acc.at[pl.ds(s * ROWS, ROWS)],
                        out_hbm.at[pl.ds(s * ROWS, ROWS)])

    @pl.when(s == 15)
    def _():
        rem = N - 15 * ROWS
        pltpu.sync_copy(acc.at[pl.ds(15 * ROWS, rem)],
                        out_hbm.at[pl.ds(15 * ROWS, rem)])


DEG_SB = 25
DEG_NSB = DEG_TILE_CHUNKS // DEG_SB
DEG_SB_EDGES = DEG_SB * CHUNK


@functools.partial(
    pl.kernel,
    mesh=_mesh,
    out_type=(jax.ShapeDtypeStruct((N, DHH), jnp.float32),
              jax.ShapeDtypeStruct((N, DHH), jnp.float32)),
    scratch_types=[
        pltpu.VMEM((DEG_SB_EDGES,), jnp.int32),
        pltpu.VMEM((CHUNK, DHH), jnp.float32),
        pltpu.VMEM_SHARED((ACC_ROWS, DHH), jnp.float32),
        pltpu.SemaphoreType.DMA,
        pltpu.SemaphoreType.DMA,
    ],
)
def _deg_sc(dst_hbm, ones_hbm, zeros_hbm, deg0_hbm, deg1_hbm,
            d_blk, ones_v, dacc, ssem0, ssem1):
    c = lax.axis_index("c")
    s = lax.axis_index("s")
    _zero_acc(zeros_hbm, dacc, s)
    pltpu.sync_copy(ones_hbm, ones_v)
    plsc.subcore_barrier()
    base0 = (c * 16 + s) * DEG_TILE_EDGES
    ssems = (ssem0, ssem1)

    def scat_wait(ssem):
        pltpu.make_async_copy(ones_v, dacc.at[d_blk.at[pl.ds(0, CHUNK)]],
                              ssem).wait()

    def body(k, carry):
        pltpu.sync_copy(dst_hbm.at[pl.ds(base0 + k * DEG_SB_EDGES,
                                         DEG_SB_EDGES)], d_blk)
        for j in range(DEG_SB):
            if j >= 2:
                scat_wait(ssems[j % 2])
            pltpu.async_copy(ones_v,
                             dacc.at[d_blk.at[pl.ds(j * CHUNK, CHUNK)]],
                             ssems[j % 2], add=True)
        scat_wait(ssems[(DEG_SB - 2) % 2])
        scat_wait(ssems[(DEG_SB - 1) % 2])
        return carry

    lax.fori_loop(0, DEG_NSB, body, 0)
    plsc.subcore_barrier()

    @pl.when(c == 0)
    def _():
        _write_out(dacc, deg0_hbm, s)

    @pl.when(c == 1)
    def _():
        _write_out(dacc, deg1_hbm, s)


SB = 25
NSB = TILE_CHUNKS // SB
SB_EDGES = SB * CHUNK


@functools.partial(
    pl.kernel,
    mesh=_mesh,
    out_type=(jax.ShapeDtypeStruct((N, DHH), jnp.float32),
              jax.ShapeDtypeStruct((N, DHH), jnp.float32)),
    scratch_types=[
        pltpu.VMEM((SB_EDGES,), jnp.int32),
        pltpu.VMEM((SB_EDGES,), jnp.int32),
        pltpu.VMEM((SB_EDGES,), jnp.int32),
        pltpu.VMEM((SB_EDGES,), jnp.int32),
        pltpu.VMEM((CHUNK, DHH), jnp.float32),
        pltpu.VMEM((CHUNK, DHH), jnp.float32),
        pltpu.VMEM((CHUNK, DHH), jnp.float32),
        pltpu.VMEM_SHARED((ACC_ROWS, DHH), jnp.float32),
        pltpu.SemaphoreType.DMA,
        pltpu.SemaphoreType.DMA,
        pltpu.SemaphoreType.DMA,
        pltpu.SemaphoreType.DMA,
        pltpu.SemaphoreType.DMA,
        pltpu.SemaphoreType.DMA,
        pltpu.SemaphoreType.DMA,
        pltpu.SemaphoreType.DMA,
    ],
)
def _prop_sc(glo_hbm, ghi_hbm, src_hbm, dst_hbm, zeros_hbm, plo_hbm, phi_hbm,
             sA, dA, sB, dB, buf0, buf1, buf2, acc,
             sem0, sem1, sem2, semA, semB, ssem0, ssem1, ssem2):
    c = lax.axis_index("c")
    s = lax.axis_index("s")
    _zero_acc(zeros_hbm, acc, s)
    plsc.subcore_barrier()
    base0 = s * TILE_EDGES

    def pipeline(gref):
        bufs = (buf0, buf1, buf2)
        sems = (sem0, sem1, sem2)
        ssems = (ssem0, ssem1, ssem2)

        def idx_fire(sb_i, s_blk, d_blk, sem):
            base = base0 + sb_i * SB_EDGES
            pltpu.async_copy(src_hbm.at[pl.ds(base, SB_EDGES)], s_blk, sem)
            pltpu.async_copy(dst_hbm.at[pl.ds(base, SB_EDGES)], d_blk, sem)

        def idx_wait(sb_i, s_blk, d_blk, sem):
            base = base0 + sb_i * SB_EDGES
            pltpu.make_async_copy(src_hbm.at[pl.ds(base, SB_EDGES)],
                                  s_blk, sem).wait()
            pltpu.make_async_copy(dst_hbm.at[pl.ds(base, SB_EDGES)],
                                  d_blk, sem).wait()

        def gather_fire(s_blk, j, buf, sem):
            pltpu.async_copy(gref.at[s_blk.at[pl.ds(j * CHUNK, CHUNK)]],
                             buf, sem)

        def scat_wait(d_blk, buf, ssem):
            pltpu.make_async_copy(buf, acc.at[d_blk.at[pl.ds(0, CHUNK)]],
                                  ssem).wait()

        def consume(s_blk, d_blk, j, buf, sem, ssem):
            pltpu.make_async_copy(gref.at[s_blk.at[pl.ds(j * CHUNK, CHUNK)]],
                                  buf, sem).wait()
            pltpu.async_copy(buf, acc.at[d_blk.at[pl.ds(j * CHUNK, CHUNK)]],
                             ssem, add=True)

        def superblock(sb_i, s_blk, d_blk):
            for j in range(SB):
                if j + 2 < SB:
                    if j >= 1:
                        scat_wait(d_blk, bufs[(j + 2) % 3], ssems[(j + 2) % 3])
                    gather_fire(s_blk, j + 2, bufs[(j + 2) % 3],
                                sems[(j + 2) % 3])
                consume(s_blk, d_blk, j, bufs[j % 3], sems[j % 3],
                        ssems[j % 3])
            scat_wait(d_blk, bufs[(SB - 3) % 3], ssems[(SB - 3) % 3])
            scat_wait(d_blk, bufs[(SB - 2) % 3], ssems[(SB - 2) % 3])
            scat_wait(d_blk, bufs[(SB - 1) % 3], ssems[(SB - 1) % 3])

        def prime(s_blk):
            gather_fire(s_blk, 0, buf0, sem0)
            gather_fire(s_blk, 1, buf1, sem1)

        idx_fire(0, sA, dA, semA)
        idx_wait(0, sA, dA, semA)
        prime(sA)

        def body(k2, carry):
            a = 2 * k2
            idx_fire(a + 1, sB, dB, semB)
            superblock(a, sA, dA)
            idx_wait(a + 1, sB, dB, semB)
            prime(sB)

            @pl.when(k2 < NSB // 2 - 1)
            def _():
                idx_fire(a + 2, sA, dA, semA)

            superblock(a + 1, sB, dB)

            @pl.when(k2 < NSB // 2 - 1)
            def _():
                idx_wait(a + 2, sA, dA, semA)
                prime(sA)

            return carry

        lax.fori_loop(0, NSB // 2, body, 0)

    @pl.when(c == 0)
    def _():
        pipeline(glo_hbm)

    @pl.when(c == 1)
    def _():
        pipeline(ghi_hbm)

    plsc.subcore_barrier()

    @pl.when(c == 0)
    def _():
        _write_out(acc, plo_hbm, s)

    @pl.when(c == 1)
    def _():
        _write_out(acc, phi_hbm, s)


def _mm1_body(d0_ref, d1_ref, x_ref, w_ref, glo_ref, ghi_ref, dinv_ref):
    deg = d0_ref[:, :1] + d1_ref[:, :1] + 1.0
    dinv = lax.rsqrt(deg)
    g = dinv * jnp.dot(x_ref[...], w_ref[...],
                       preferred_element_type=jnp.float32)
    glo_ref[...] = g[:, :DHH]
    ghi_ref[...] = g[:, DHH:]
    dinv_ref[...] = dinv


def _mid_body(plo_ref, phi_ref, glo_ref, ghi_ref, dinv_ref, b_ref, w_ref,
              olo_ref, ohi_ref):
    dinv = dinv_ref[...]
    pg = jnp.concatenate([plo_ref[...] + glo_ref[...],
                          phi_ref[...] + ghi_ref[...]], axis=1)
    h = jnp.maximum(dinv * pg + b_ref[...], 0.0)
    g = dinv * jnp.dot(h, w_ref[...], preferred_element_type=jnp.float32)
    olo_ref[...] = g[:, :DHH]
    ohi_ref[...] = g[:, DHH:]


def _fin_body(plo_ref, phi_ref, glo_ref, ghi_ref, dinv_ref, b_ref, batch_ref,
              gamma_ref, beta_ref, out_ref, pool_acc, cnt_acc):
    i = pl.program_id(0)

    @pl.when(i == 0)
    def _():
        pool_acc[...] = jnp.zeros_like(pool_acc)
        cnt_acc[...] = jnp.zeros_like(cnt_acc)

    dinv = dinv_ref[...]
    pg = jnp.concatenate([plo_ref[...] + glo_ref[...],
                          phi_ref[...] + ghi_ref[...]], axis=1)
    h = jnp.maximum(dinv * pg + b_ref[...], 0.0)
    ids = lax.broadcasted_iota(jnp.int32, (G_GRAPHS, BLK), 0)
    onehot = (ids == batch_ref[0]).astype(jnp.float32)
    pool_acc[...] += jnp.dot(onehot, h, preferred_element_type=jnp.float32)
    cnt_acc[...] += jnp.sum(onehot, axis=1, keepdims=True)

    @pl.when(i == GRID - 1)
    def _():
        pooled = pool_acc[...] / jnp.maximum(cnt_acc[...], 1.0)
        mu = jnp.mean(pooled, axis=-1, keepdims=True)
        var = jnp.mean((pooled - mu) ** 2, axis=-1, keepdims=True)
        out_ref[...] = ((pooled - mu) * lax.rsqrt(var + 1e-5)
                        * gamma_ref[...] + beta_ref[...])


_ROW_SPEC = pl.BlockSpec((BLK, DHH), lambda i: (i, 0))
_DINV_SPEC = pl.BlockSpec((BLK, 1), lambda i: (i, 0))
_VEC_SPEC = pl.BlockSpec((1, D_H), lambda i: (0, 0))


def _mm1(d0, d1, x, w1):
    return pl.pallas_call(
        _mm1_body,
        grid=(GRID,),
        in_specs=[
            _ROW_SPEC,
            _ROW_SPEC,
            pl.BlockSpec((BLK, D_IN), lambda i: (i, 0)),
            pl.BlockSpec((D_IN, D_H), lambda i: (0, 0)),
        ],
        out_specs=[_ROW_SPEC, _ROW_SPEC, _DINV_SPEC],
        out_shape=[
            jax.ShapeDtypeStruct((N, DHH), jnp.float32),
            jax.ShapeDtypeStruct((N, DHH), jnp.float32),
            jax.ShapeDtypeStruct((N, 1), jnp.float32),
        ],
    )(d0, d1, x, w1)


def _mid(plo, phi, glo, ghi, dinv, b, w):
    return pl.pallas_call(
        _mid_body,
        grid=(GRID,),
        in_specs=[
            _ROW_SPEC, _ROW_SPEC, _ROW_SPEC, _ROW_SPEC, _DINV_SPEC,
            _VEC_SPEC,
            pl.BlockSpec((D_H, D_H), lambda i: (0, 0)),
        ],
        out_specs=[_ROW_SPEC, _ROW_SPEC],
        out_shape=[
            jax.ShapeDtypeStruct((N, DHH), jnp.float32),
            jax.ShapeDtypeStruct((N, DHH), jnp.float32),
        ],
    )(plo, phi, glo, ghi, dinv, b, w)


def _fin(plo, phi, glo, ghi, dinv, b, batch_r, gamma, beta):
    return pl.pallas_call(
        _fin_body,
        grid=(GRID,),
        in_specs=[
            _ROW_SPEC, _ROW_SPEC, _ROW_SPEC, _ROW_SPEC, _DINV_SPEC,
            _VEC_SPEC,
            pl.BlockSpec((1, 1, BLK), lambda i: (i, 0, 0)),
            _VEC_SPEC, _VEC_SPEC,
        ],
        out_specs=pl.BlockSpec((G_GRAPHS, D_H), lambda i: (0, 0)),
        out_shape=jax.ShapeDtypeStruct((G_GRAPHS, D_H), jnp.float32),
        scratch_shapes=[
            pltpu.VMEM((G_GRAPHS, D_H), jnp.float32),
            pltpu.VMEM((G_GRAPHS, 1), jnp.float32),
        ],
    )(plo, phi, glo, ghi, dinv, b, batch_r, gamma, beta)


def kernel(x, edge_index, edge_weight, batch, W1, b1, W2, b2, W3, b3,
           gamma, beta):
    src = edge_index[0]
    dst = edge_index[1]
    ones128 = jnp.ones((CHUNK, DHH), jnp.float32)
    zeros128 = jnp.zeros((ROWS, DHH), jnp.float32)

    d0, d1 = _deg_sc(dst, ones128, zeros128)
    glo1, ghi1, dinv = _mm1(d0, d1, x, W1)
    plo1, phi1 = _prop_sc(glo1, ghi1, src, dst, zeros128)
    glo2, ghi2 = _mid(plo1, phi1, glo1, ghi1, dinv, b1.reshape(1, D_H), W2)
    plo2, phi2 = _prop_sc(glo2, ghi2, src, dst, zeros128)
    glo3, ghi3 = _mid(plo2, phi2, glo2, ghi2, dinv, b2.reshape(1, D_H), W3)
    plo3, phi3 = _prop_sc(glo3, ghi3, src, dst, zeros128)
    return _fin(plo3, phi3, glo3, ghi3, dinv, b3.reshape(1, D_H),
                batch.reshape(GRID, 1, BLK),
                gamma.reshape(1, D_H), beta.reshape(1, D_H))
